# baseline (speedup 1.0000x reference)
import jax
import jax.numpy as jnp
from jax import lax
from jax.experimental import pallas as pl
from jax.experimental.pallas import tpu as pltpu

_UPDATE_SCALE = 1.0
_DIST_THRESHOLD = 3.0
_BIG = 1e4
_INIT = 1e9
_PAD = 4

_EXT = {"z": (0.0, 0.0), "v": (1.0, 0.0), "h": (0.0, 1.0)}


def _offsets(tp, tg):
    apr, apc = _EXT[tp]
    agr, agc = _EXT[tg]
    out = []
    for di in range(-_PAD - 1, _PAD + 2):
        for dj in range(-_PAD - 1, _PAD + 2):
            gr = max(0.0, di - apr, -di - agr)
            gc = max(0.0, dj - apc, -dj - agc)
            if gr * gr + gc * gc <= 9.5:
                assert abs(di) <= _PAD and abs(dj) <= _PAD
                out.append((di, dj))
    return out


_OFFS = {(tp, tg): _offsets(tp, tg) for tp in ("v", "h") for tg in ("z", "v", "h")}


def _nn_body(pred_ref, gt_ref,
             dotv_ref, doth_ref, pixv_ref, pixh_ref,
             rz_ref, cz_ref, rv_ref, cv_ref, rh_ref, ch_ref):
    P = pred_ref[0]
    G = gt_ref[0]
    ii = lax.broadcasted_iota(jnp.int32, (64, 64), 0).astype(jnp.float32)
    jj = lax.broadcasted_iota(jnp.int32, (64, 64), 1).astype(jnp.float32)

    big72 = jnp.full((72, 72), _BIG, jnp.float32)
    vz = G == 0.0
    rz_ref[...] = big72
    cz_ref[...] = big72
    rz_ref[4:68, 4:68] = jnp.where(vz, ii, _BIG)
    cz_ref[4:68, 4:68] = jnp.where(vz, jj, _BIG)

    g1 = G[:63, :]
    g2 = G[1:, :]
    av = jnp.abs(g1) / (jnp.abs(g1) + jnp.abs(g2) + 1e-8)
    vv = (g1 * g2) < 0
    rv_ref[...] = big72
    cv_ref[...] = big72
    rv_ref[4:67, 4:68] = jnp.where(vv, ii[:63, :] + av, _BIG)
    cv_ref[4:67, 4:68] = jnp.where(vv, jj[:63, :], _BIG)

    h1 = G[:, :63]
    h2 = G[:, 1:]
    ah = jnp.abs(h1) / (jnp.abs(h1) + jnp.abs(h2) + 1e-8)
    vh = (h1 * h2) < 0
    rh_ref[...] = big72
    ch_ref[...] = big72
    rh_ref[4:68, 4:67] = jnp.where(vh, ii[:, :63], _BIG)
    ch_ref[4:68, 4:67] = jnp.where(vh, jj[:, :63] + ah, _BIG)

    zrow = jnp.zeros((1, 64), jnp.float32)
    zcol = jnp.zeros((64, 1), jnp.float32)
    p1 = P[:63, :]
    p2 = P[1:, :]
    apv = jnp.abs(p1) / (jnp.abs(p1) + jnp.abs(p2) + 1e-8)
    rp_v = ii + jnp.concatenate([apv, zrow], axis=0)
    cp_v = jj
    validv = jnp.concatenate(
        [jnp.where((p1 * p2) < 0, 1.0, 0.0).astype(jnp.float32), zrow], axis=0)
    q1 = P[:, :63]
    q2 = P[:, 1:]
    aph = jnp.abs(q1) / (jnp.abs(q1) + jnp.abs(q2) + 1e-8)
    rp_h = ii
    cp_h = jj + jnp.concatenate([aph, zcol], axis=1)
    validh = jnp.concatenate(
        [jnp.where((q1 * q2) < 0, 1.0, 0.0).astype(jnp.float32), zcol], axis=1)

    nr = jnp.concatenate([P[1:2] - P[0:1], 0.5 * (P[2:] - P[:-2]),
                          P[63:64] - P[62:63]], axis=0)
    nc = jnp.concatenate([P[:, 1:2] - P[:, 0:1], 0.5 * (P[:, 2:] - P[:, :-2]),
                          P[:, 63:64] - P[:, 62:63]], axis=1)
    P_dn = jnp.concatenate([P[1:], P[63:64]], axis=0)
    P_rt = jnp.concatenate([P[:, 1:], P[:, 63:64]], axis=1)
    nr_dn = jnp.concatenate([nr[1:], nr[63:64]], axis=0)
    nc_dn = jnp.concatenate([nc[1:], nc[63:64]], axis=0)
    nr_rt = jnp.concatenate([nr[:, 1:], nr[:, 63:64]], axis=1)
    nc_rt = jnp.concatenate([nc[:, 1:], nc[:, 63:64]], axis=1)

    gt_grids = (("z", rz_ref, cz_ref), ("v", rv_ref, cv_ref), ("h", rh_ref, ch_ref))

    def scan(tp, rp, cp):
        best = jnp.full((64, 64), _INIT, jnp.float32)
        bdr = jnp.zeros((64, 64), jnp.float32)
        bdc = jnp.zeros((64, 64), jnp.float32)
        for tg, rg_ref, cg_ref in gt_grids:
            for (di, dj) in _OFFS[(tp, tg)]:
                rw = rg_ref[4 + di:68 + di, 4 + dj:68 + dj]
                cw = cg_ref[4 + di:68 + di, 4 + dj:68 + dj]
                dr = rw - rp
                dc = cw - cp
                dist = jnp.sqrt(dr * dr + dc * dc)
                upd = dist < best
                best = jnp.where(upd, dist, best)
                bdr = jnp.where(upd, dr, bdr)
                bdc = jnp.where(upd, dc, bdc)
        return best, bdr, bdc

    def finish(best, bdr, bdc, ar, n0r, n0c, n1r, n1c, s0, s1, valid):
        inr = n0r * (1 - ar) + n1r * ar
        inc = n0c * (1 - ar) + n1c * ar
        norm = jnp.sqrt(inr * inr + inc * inc) + 1e-08
        inr = inr / norm
        inc = inc / norm
        mask = jnp.where(best <= _DIST_THRESHOLD, 1.0, 0.0).astype(jnp.float32)
        dot = (bdr * inr + bdc * inc) * _UPDATE_SCALE
        dot = dot * mask * valid
        bs = s0 * (1 - ar) + s1 * ar
        return dot, bs * valid

    bv, drv, dcv = scan("v", rp_v, cp_v)
    ar_v = rp_v - ii
    dot_v, pix_v = finish(bv, drv, dcv, ar_v, nr, nc, nr_dn, nc_dn, P, P_dn, validv)
    dotv_ref[0] = dot_v
    pixv_ref[0] = pix_v

    bh, drh, dch = scan("h", rp_h, cp_h)
    ac_h = cp_h - jj
    dot_h, pix_h = finish(bh, drh, dch, ac_h, nr, nc, nr_rt, nc_rt, P, P_rt, validh)
    doth_ref[0] = dot_h
    pixh_ref[0] = pix_h


@jax.jit
def _nn_search(pred_sdf, gt_sdf):
    B = pred_sdf.shape[0]
    spec = pl.BlockSpec((1, 64, 64), lambda b: (b, 0, 0))
    out = jax.ShapeDtypeStruct((B, 64, 64), jnp.float32)
    return pl.pallas_call(
        _nn_body,
        grid=(B,),
        in_specs=[spec, spec],
        out_specs=[spec] * 4,
        out_shape=[out] * 4,
        scratch_shapes=[pltpu.VMEM((72, 72), jnp.float32)] * 6,
    )(pred_sdf, gt_sdf)


def _extract_zero_crossings(sdf, eps=1e-08):
    H, W = sdf.shape
    v1, v2 = sdf[:-1, :], sdf[1:, :]
    mask_v = ((v1 * v2) < 0).reshape(-1)
    alpha_v = jnp.abs(v1) / (jnp.abs(v1) + jnp.abs(v2) + eps)
    rs_v = jnp.arange(H - 1, dtype=jnp.float32)[:, None] + alpha_v
    cs_v = jnp.broadcast_to(jnp.arange(W, dtype=jnp.float32)[None, :], (H - 1, W))
    pts_v = jnp.stack((rs_v.reshape(-1), cs_v.reshape(-1)), axis=1)
    h1, h2 = sdf[:, :-1], sdf[:, 1:]
    mask_h = ((h1 * h2) < 0).reshape(-1)
    alpha_h = jnp.abs(h1) / (jnp.abs(h1) + jnp.abs(h2) + eps)
    rs_h = jnp.broadcast_to(jnp.arange(H, dtype=jnp.float32)[:, None], (H, W - 1))
    cs_h = jnp.arange(W - 1, dtype=jnp.float32)[None, :] + alpha_h
    pts_h = jnp.stack((rs_h.reshape(-1), cs_h.reshape(-1)), axis=1)
    mask_z = (sdf == 0).reshape(-1)
    rz = jnp.broadcast_to(jnp.arange(H, dtype=jnp.float32)[:, None], (H, W)).reshape(-1)
    cz = jnp.broadcast_to(jnp.arange(W, dtype=jnp.float32)[None, :], (H, W)).reshape(-1)
    pts_z = jnp.stack((rz, cz), axis=1)
    pts = jnp.concatenate((pts_z, pts_v, pts_h), axis=0)
    valid = jnp.concatenate((mask_z, mask_v, mask_h), axis=0)
    return pts, valid


def _chamfer_grad(pred2d, pred_zc, dot):
    H, W = pred2d.shape
    r, c = pred_zc[:, 0], pred_zc[:, 1]
    r0 = jnp.clip(jnp.floor(r).astype(jnp.int32), 0, H - 1)
    c0 = jnp.clip(jnp.floor(c).astype(jnp.int32), 0, W - 1)
    r1 = jnp.clip(r0 + 1, 0, H - 1)
    c1 = jnp.clip(c0 + 1, 0, W - 1)
    ar = r - r0.astype(jnp.float32)
    ac = c - c0.astype(jnp.float32)
    w00 = (1 - ar) * (1 - ac)
    w01 = (1 - ar) * ac
    w10 = ar * (1 - ac)
    w11 = ar * ac
    idx00 = r0 * W + c0
    idx01 = r0 * W + c1
    idx10 = r1 * W + c0
    idx11 = r1 * W + c1
    indices = jnp.concatenate((idx00, idx01, idx10, idx11), axis=0)
    contribs = jnp.concatenate((dot * w00, dot * w01, dot * w10, dot * w11), axis=0)
    dflat = jnp.zeros(H * W, dtype=pred2d.dtype).at[indices].add(contribs)
    return dflat.reshape(H, W)


def kernel(pred_sdf, gt_sdf):
    B = pred_sdf.shape[0]
    dotv, doth, pixv, pixh = _nn_search(pred_sdf, gt_sdf)
    inject_terms = []
    pixel_terms = []
    zfill = jnp.zeros((4096,), jnp.float32)
    for b in range(B):
        pred2d = pred_sdf[b]
        pred_zc, _pred_valid = _extract_zero_crossings(pred2d)
        dot = jnp.concatenate(
            (zfill, dotv[b, :63, :].reshape(-1), doth[b, :, :63].reshape(-1)))
        dSDF = _chamfer_grad(pred2d, pred_zc, dot)
        inject_terms.append(jnp.sum(pred2d * dSDF))
        pixel_terms.append(jnp.sum(pixv[b]) + jnp.sum(pixh[b]))
    inject = jnp.stack(inject_terms).mean()
    pixel = jnp.stack(pixel_terms).mean()
    return 1.0 * inject + 1.0 * pixel
